# baseline (speedup 1.0000x reference)
import functools
import jax
import jax.numpy as jnp
from jax import lax
from jax.experimental import pallas as pl
from jax.experimental.pallas import tpu as pltpu
from jax.experimental.pallas import tpu_sc as plsc

D_MODEL = 768
N_EXP = 8
D_EXP = 2048
T = 2048
TM = 128
NT = 40
NSLOT = NT * TM
DAUG = D_MODEL + 128
NW = 32
TPW = T // NW
CHUNK = 256


def _router_body(x_ref, wg_ref, xaug_ref, slots_ref, te_ref, m_scr, rank_scr):
    x = x_ref[...]
    logits = jnp.dot(x, wg_ref[...], preferred_element_type=jnp.float32)
    cols = jax.lax.broadcasted_iota(jnp.int32, logits.shape, 1)
    big = jnp.int32(N_EXP)
    v1 = jnp.max(logits, axis=1, keepdims=True)
    i1 = jnp.min(jnp.where(logits == v1, cols, big), axis=1, keepdims=True)
    l2 = jnp.where(cols == i1, -jnp.inf, logits)
    v2 = jnp.max(l2, axis=1, keepdims=True)
    i2 = jnp.min(jnp.where(l2 == v2, cols, big), axis=1, keepdims=True)
    e2 = jnp.exp(v2 - v1)
    denom = 1.0 + e2
    g1 = 1.0 / denom
    g2 = e2 / denom

    m = ((cols == i1) | (cols == i2)).astype(jnp.float32)
    m_scr[...] = m

    ri = jax.lax.broadcasted_iota(jnp.int32, (CHUNK, CHUNK), 0)
    ci = jax.lax.broadcasted_iota(jnp.int32, (CHUNK, CHUNK), 1)
    ltri = (ri > ci).astype(jnp.float32)

    def chunk_body(c, carry):
        mc = m_scr[pl.ds(c * CHUNK, CHUNK), :]
        rank_scr[pl.ds(c * CHUNK, CHUNK), :] = (
            jnp.dot(ltri, mc, preferred_element_type=jnp.float32) + carry)
        return carry + jnp.sum(mc, axis=0, keepdims=True)

    counts = jax.lax.fori_loop(0, T // CHUNK, chunk_body,
                               jnp.zeros((1, N_EXP), jnp.float32))

    ntiles = jnp.floor((counts + (TM - 1)) / TM)
    ei = jax.lax.broadcasted_iota(jnp.int32, (N_EXP, N_EXP), 0)
    ej = jax.lax.broadcasted_iota(jnp.int32, (N_EXP, N_EXP), 1)
    strict = (ei < ej).astype(jnp.float32)
    tile_base = jnp.dot(ntiles, strict, preferred_element_type=jnp.float32)
    offset = tile_base * TM

    slot_full = offset + rank_scr[...]
    slot1 = jnp.sum(jnp.where(cols == i1, slot_full, 0.0), axis=1, keepdims=True)
    slot2 = jnp.sum(jnp.where(cols == i2, slot_full, 0.0), axis=1, keepdims=True)

    sc = jax.lax.broadcasted_iota(jnp.int32, (T, 128), 1)
    s1b = jnp.broadcast_to(slot1, (T, 128))
    s2b = jnp.broadcast_to(slot2, (T, 128))
    slots_ref[...] = jnp.where(sc == 0, s1b, jnp.where(sc == 1, s2b, 0.0)
                               ).astype(jnp.int32)

    tj = jax.lax.broadcasted_iota(jnp.int32, (NT, N_EXP), 0).astype(jnp.float32)
    eid = jax.lax.broadcasted_iota(jnp.int32, (NT, N_EXP), 1).astype(jnp.float32)
    tb = jnp.broadcast_to(tile_base, (NT, N_EXP))
    ntb = jnp.broadcast_to(ntiles, (NT, N_EXP))
    ind = ((tj >= tb) & (tj < tb + ntb)).astype(jnp.float32)
    te = jnp.sum(ind * eid, axis=1, keepdims=True)
    te_ref[...] = jnp.broadcast_to(te, (NT, 128)).astype(jnp.int32)

    xaug_ref[0] = jnp.concatenate(
        [x, jnp.broadcast_to(g1, (T, 128))], axis=1)
    xaug_ref[1] = jnp.concatenate(
        [x, jnp.broadcast_to(g2, (T, 128))], axis=1)


@jax.jit
def _router(x2d, w_gate):
    return pl.pallas_call(
        _router_body,
        in_specs=[pl.BlockSpec((T, D_MODEL), lambda: (0, 0)),
                  pl.BlockSpec((D_MODEL, N_EXP), lambda: (0, 0))],
        out_specs=[pl.BlockSpec((2, T, DAUG), lambda: (0, 0, 0)),
                   pl.BlockSpec((T, 128), lambda: (0, 0)),
                   pl.BlockSpec((NT, 128), lambda: (0, 0))],
        out_shape=[jax.ShapeDtypeStruct((2, T, DAUG), jnp.float32),
                   jax.ShapeDtypeStruct((T, 128), jnp.int32),
                   jax.ShapeDtypeStruct((NT, 128), jnp.int32)],
        scratch_shapes=[pltpu.VMEM((T, N_EXP), jnp.float32),
                        pltpu.VMEM((T, N_EXP), jnp.float32)],
    )(x2d, w_gate)


@functools.cache
def _sc_dispatch():
    mesh = plsc.VectorSubcoreMesh(core_axis_name="c", subcore_axis_name="s")

    @functools.partial(
        pl.kernel, mesh=mesh,
        out_type=jax.ShapeDtypeStruct((NSLOT, DAUG), jnp.float32),
        scratch_types=[pltpu.VMEM((TPW,), jnp.int32),
                       pltpu.VMEM((TPW,), jnp.int32),
                       pltpu.VMEM((TPW, DAUG), jnp.float32),
                       pltpu.VMEM((TPW, DAUG), jnp.float32),
                       pltpu.SemaphoreType.DMA,
                       pltpu.SemaphoreType.DMA],
    )
    def _dispatch(xaug_hbm, slot1_hbm, slot2_hbm, xg_hbm, idx1_v, idx2_v,
                  buf1_v, buf2_v, sem1, sem2):
        wid = lax.axis_index("s") * 2 + lax.axis_index("c")
        base = wid * TPW
        pltpu.sync_copy(slot1_hbm.at[wid], idx1_v)
        pltpu.sync_copy(xaug_hbm.at[0, pl.ds(base, TPW)], buf1_v)
        cp1 = pltpu.async_copy(buf1_v, xg_hbm.at[idx1_v], sem1)
        pltpu.sync_copy(slot2_hbm.at[wid], idx2_v)
        pltpu.sync_copy(xaug_hbm.at[1, pl.ds(base, TPW)], buf2_v)
        cp2 = pltpu.async_copy(buf2_v, xg_hbm.at[idx2_v], sem2)
        cp1.wait()
        cp2.wait()

    return _dispatch


def _gmm_body(te_ref, xg_ref, w1_ref, b1_ref, w2_ref, b2_ref, y_ref):
    xg = xg_ref[...]
    x = xg[:, :D_MODEL].astype(jnp.bfloat16)
    g = xg[:, D_MODEL:D_MODEL + 1]
    h = jnp.maximum(jnp.dot(x, w1_ref[0], preferred_element_type=jnp.float32)
                    + b1_ref[0], 0.0)
    y_ref[...] = (jnp.dot(h.astype(jnp.bfloat16), w2_ref[0],
                          preferred_element_type=jnp.float32)
                  + b2_ref[0]) * g


@jax.jit
def _gmm(te, xg, w1, b1r, w2, b2r):
    grid_spec = pltpu.PrefetchScalarGridSpec(
        num_scalar_prefetch=1,
        grid=(NT,),
        in_specs=[
            pl.BlockSpec((TM, DAUG), lambda j, te: (j, 0)),
            pl.BlockSpec((1, D_MODEL, D_EXP), lambda j, te: (te[j], 0, 0)),
            pl.BlockSpec((1, 1, D_EXP), lambda j, te: (te[j], 0, 0)),
            pl.BlockSpec((1, D_EXP, D_MODEL), lambda j, te: (te[j], 0, 0)),
            pl.BlockSpec((1, 1, D_MODEL), lambda j, te: (te[j], 0, 0)),
        ],
        out_specs=pl.BlockSpec((TM, D_MODEL), lambda j, te: (j, 0)),
    )
    return pl.pallas_call(
        _gmm_body,
        grid_spec=grid_spec,
        out_shape=jax.ShapeDtypeStruct((NSLOT, D_MODEL), jnp.float32),
    )(te, xg, w1, b1r, w2, b2r)


@functools.cache
def _sc_combine():
    mesh = plsc.VectorSubcoreMesh(core_axis_name="c", subcore_axis_name="s")

    @functools.partial(
        pl.kernel, mesh=mesh,
        out_type=jax.ShapeDtypeStruct((T, D_MODEL), jnp.float32),
        scratch_types=[pltpu.VMEM((TPW,), jnp.int32),
                       pltpu.VMEM((TPW,), jnp.int32),
                       pltpu.VMEM((TPW, D_MODEL), jnp.float32),
                       pltpu.VMEM((TPW, D_MODEL), jnp.float32),
                       pltpu.SemaphoreType.DMA],
    )
    def _combine(y_hbm, slot1_hbm, slot2_hbm, out_hbm, i1_v, i2_v, b1_v, b2_v,
                 sem):
        wid = lax.axis_index("s") * 2 + lax.axis_index("c")
        base = wid * TPW
        pltpu.sync_copy(slot1_hbm.at[wid], i1_v)
        pltpu.sync_copy(slot2_hbm.at[wid], i2_v)
        pltpu.async_copy(y_hbm.at[i1_v], b1_v, sem).wait()
        pltpu.async_copy(y_hbm.at[i2_v], b2_v, sem).wait()

        def row(i, _):
            for c in range(D_MODEL // 16):
                sl = pl.ds(c * 16, 16)
                b1_v[i, sl] = b1_v[i, sl] + b2_v[i, sl]
            return 0

        jax.lax.fori_loop(0, TPW, row, 0)
        pltpu.sync_copy(b1_v, out_hbm.at[pl.ds(base, TPW)])

    return _combine


def kernel(x, w_gate, w1, b1, w2, b2):
    x2d = x.reshape(T, D_MODEL)
    xaug, slots, te_w = _router(x2d, w_gate)
    slot1 = slots[:, 0].reshape(NW, TPW)
    slot2 = slots[:, 1].reshape(NW, TPW)
    te = te_w[:, 0]
    xg = _sc_dispatch()(xaug, slot1, slot2)
    y = _gmm(te, xg, w1.astype(jnp.bfloat16), b1.reshape(N_EXP, 1, D_EXP),
             w2.astype(jnp.bfloat16), b2.reshape(N_EXP, 1, D_MODEL))
    out = _sc_combine()(y, slot1, slot2)
    return out.reshape(x.shape)


# speedup vs baseline: 1.1998x; 1.1998x over previous
import functools
import jax
import jax.numpy as jnp
from jax import lax
from jax.experimental import pallas as pl
from jax.experimental.pallas import tpu as pltpu
from jax.experimental.pallas import tpu_sc as plsc

D_MODEL = 768
N_EXP = 8
D_EXP = 2048
T = 2048
TM = 128
NT = 40
NSLOT = NT * TM
DAUG = D_MODEL + 128
NW = 32
TPW = T // NW
CHUNK = 256


def _router_body(x_ref, wg_ref, xaug_ref, slots_ref, te_ref, m_scr, rank_scr):
    x = x_ref[...]
    logits = jnp.dot(x, wg_ref[...], preferred_element_type=jnp.float32)
    cols = jax.lax.broadcasted_iota(jnp.int32, logits.shape, 1)
    big = jnp.int32(N_EXP)
    v1 = jnp.max(logits, axis=1, keepdims=True)
    i1 = jnp.min(jnp.where(logits == v1, cols, big), axis=1, keepdims=True)
    l2 = jnp.where(cols == i1, -jnp.inf, logits)
    v2 = jnp.max(l2, axis=1, keepdims=True)
    i2 = jnp.min(jnp.where(l2 == v2, cols, big), axis=1, keepdims=True)
    e2 = jnp.exp(v2 - v1)
    denom = 1.0 + e2
    g1 = 1.0 / denom
    g2 = e2 / denom

    m = ((cols == i1) | (cols == i2)).astype(jnp.float32)
    m_scr[...] = m

    ri = jax.lax.broadcasted_iota(jnp.int32, (CHUNK, CHUNK), 0)
    ci = jax.lax.broadcasted_iota(jnp.int32, (CHUNK, CHUNK), 1)
    ltri = (ri > ci).astype(jnp.float32)

    def chunk_body(c, carry):
        mc = m_scr[pl.ds(c * CHUNK, CHUNK), :]
        rank_scr[pl.ds(c * CHUNK, CHUNK), :] = (
            jnp.dot(ltri, mc, preferred_element_type=jnp.float32) + carry)
        return carry + jnp.sum(mc, axis=0, keepdims=True)

    counts = jax.lax.fori_loop(0, T // CHUNK, chunk_body,
                               jnp.zeros((1, N_EXP), jnp.float32))

    ntiles = jnp.floor((counts + (TM - 1)) / TM)
    ei = jax.lax.broadcasted_iota(jnp.int32, (N_EXP, N_EXP), 0)
    ej = jax.lax.broadcasted_iota(jnp.int32, (N_EXP, N_EXP), 1)
    strict = (ei < ej).astype(jnp.float32)
    tile_base = jnp.dot(ntiles, strict, preferred_element_type=jnp.float32)
    offset = tile_base * TM

    slot_full = offset + rank_scr[...]
    slot1 = jnp.sum(jnp.where(cols == i1, slot_full, 0.0), axis=1, keepdims=True)
    slot2 = jnp.sum(jnp.where(cols == i2, slot_full, 0.0), axis=1, keepdims=True)

    sc = jax.lax.broadcasted_iota(jnp.int32, (T, 128), 1)
    s1b = jnp.broadcast_to(slot1, (T, 128))
    s2b = jnp.broadcast_to(slot2, (T, 128))
    slots_ref[...] = jnp.where(sc == 0, s1b, jnp.where(sc == 1, s2b, 0.0)
                               ).astype(jnp.int32)

    tj = jax.lax.broadcasted_iota(jnp.int32, (NT, N_EXP), 0).astype(jnp.float32)
    eid = jax.lax.broadcasted_iota(jnp.int32, (NT, N_EXP), 1).astype(jnp.float32)
    tb = jnp.broadcast_to(tile_base, (NT, N_EXP))
    ntb = jnp.broadcast_to(ntiles, (NT, N_EXP))
    ind = ((tj >= tb) & (tj < tb + ntb)).astype(jnp.float32)
    te = jnp.sum(ind * eid, axis=1, keepdims=True)
    te_ref[...] = jnp.broadcast_to(te, (NT, 128)).astype(jnp.int32)

    xaug_ref[0] = jnp.concatenate(
        [x, jnp.broadcast_to(g1, (T, 128))], axis=1)
    xaug_ref[1] = jnp.concatenate(
        [x, jnp.broadcast_to(g2, (T, 128))], axis=1)


@jax.jit
def _router(x2d, w_gate):
    return pl.pallas_call(
        _router_body,
        in_specs=[pl.BlockSpec((T, D_MODEL), lambda: (0, 0)),
                  pl.BlockSpec((D_MODEL, N_EXP), lambda: (0, 0))],
        out_specs=[pl.BlockSpec((2, T, DAUG), lambda: (0, 0, 0)),
                   pl.BlockSpec((T, 128), lambda: (0, 0)),
                   pl.BlockSpec((NT, 128), lambda: (0, 0))],
        out_shape=[jax.ShapeDtypeStruct((2, T, DAUG), jnp.float32),
                   jax.ShapeDtypeStruct((T, 128), jnp.int32),
                   jax.ShapeDtypeStruct((NT, 128), jnp.int32)],
        scratch_shapes=[pltpu.VMEM((T, N_EXP), jnp.float32),
                        pltpu.VMEM((T, N_EXP), jnp.float32)],
    )(x2d, w_gate)


@functools.cache
def _sc_dispatch():
    mesh = plsc.VectorSubcoreMesh(core_axis_name="c", subcore_axis_name="s")

    @functools.partial(
        pl.kernel, mesh=mesh,
        out_type=jax.ShapeDtypeStruct((NSLOT, DAUG), jnp.float32),
        scratch_types=[pltpu.VMEM((TPW,), jnp.int32),
                       pltpu.VMEM((TPW,), jnp.int32),
                       pltpu.VMEM((TPW, DAUG), jnp.float32),
                       pltpu.VMEM((TPW, DAUG), jnp.float32),
                       pltpu.SemaphoreType.DMA,
                       pltpu.SemaphoreType.DMA],
    )
    def _dispatch(xaug_hbm, slot1_hbm, slot2_hbm, xg_hbm, idx1_v, idx2_v,
                  buf1_v, buf2_v, sem1, sem2):
        wid = lax.axis_index("s") * 2 + lax.axis_index("c")
        base = wid * TPW
        pltpu.sync_copy(slot1_hbm.at[wid], idx1_v)
        pltpu.sync_copy(xaug_hbm.at[0, pl.ds(base, TPW)], buf1_v)
        cp1 = pltpu.async_copy(buf1_v, xg_hbm.at[idx1_v], sem1)
        pltpu.sync_copy(slot2_hbm.at[wid], idx2_v)
        pltpu.sync_copy(xaug_hbm.at[1, pl.ds(base, TPW)], buf2_v)
        cp2 = pltpu.async_copy(buf2_v, xg_hbm.at[idx2_v], sem2)
        cp1.wait()
        cp2.wait()

    return _dispatch


def _gmm_body(te_ref, xg_ref, w1_ref, b1_ref, w2_ref, b2_ref, y_ref):
    xg = xg_ref[...]
    x = xg[:, :D_MODEL]
    g = xg[:, D_MODEL:D_MODEL + 1]
    h = jnp.maximum(jnp.dot(x, w1_ref[0], preferred_element_type=jnp.float32)
                    + b1_ref[0], 0.0)
    y_ref[...] = (jnp.dot(h, w2_ref[0], preferred_element_type=jnp.float32)
                  + b2_ref[0]) * g


@jax.jit
def _gmm(te, xg, w1, b1r, w2, b2r):
    grid_spec = pltpu.PrefetchScalarGridSpec(
        num_scalar_prefetch=1,
        grid=(NT,),
        in_specs=[
            pl.BlockSpec((TM, DAUG), lambda j, te: (j, 0)),
            pl.BlockSpec((1, D_MODEL, D_EXP), lambda j, te: (te[j], 0, 0)),
            pl.BlockSpec((1, 1, D_EXP), lambda j, te: (te[j], 0, 0)),
            pl.BlockSpec((1, D_EXP, D_MODEL), lambda j, te: (te[j], 0, 0)),
            pl.BlockSpec((1, 1, D_MODEL), lambda j, te: (te[j], 0, 0)),
        ],
        out_specs=pl.BlockSpec((TM, D_MODEL), lambda j, te: (j, 0)),
    )
    return pl.pallas_call(
        _gmm_body,
        grid_spec=grid_spec,
        out_shape=jax.ShapeDtypeStruct((NSLOT, D_MODEL), jnp.float32),
    )(te, xg, w1, b1r, w2, b2r)


@functools.cache
def _sc_combine():
    mesh = plsc.VectorSubcoreMesh(core_axis_name="c", subcore_axis_name="s")

    @functools.partial(
        pl.kernel, mesh=mesh,
        out_type=jax.ShapeDtypeStruct((T, D_MODEL), jnp.float32),
        scratch_types=[pltpu.VMEM((TPW,), jnp.int32),
                       pltpu.VMEM((TPW,), jnp.int32),
                       pltpu.VMEM((TPW, D_MODEL), jnp.float32),
                       pltpu.VMEM((TPW, D_MODEL), jnp.float32),
                       pltpu.SemaphoreType.DMA],
    )
    def _combine(y_hbm, slot1_hbm, slot2_hbm, out_hbm, i1_v, i2_v, b1_v, b2_v,
                 sem):
        wid = lax.axis_index("s") * 2 + lax.axis_index("c")
        base = wid * TPW
        pltpu.sync_copy(slot1_hbm.at[wid], i1_v)
        pltpu.sync_copy(slot2_hbm.at[wid], i2_v)
        pltpu.async_copy(y_hbm.at[i1_v], b1_v, sem).wait()
        pltpu.async_copy(y_hbm.at[i2_v], b2_v, sem).wait()

        def row(i, _):
            for c in range(D_MODEL // 16):
                sl = pl.ds(c * 16, 16)
                b1_v[i, sl] = b1_v[i, sl] + b2_v[i, sl]
            return 0

        jax.lax.fori_loop(0, TPW, row, 0)
        pltpu.sync_copy(b1_v, out_hbm.at[pl.ds(base, TPW)])

    return _combine


def kernel(x, w_gate, w1, b1, w2, b2):
    x2d = x.reshape(T, D_MODEL)
    xaug, slots, te_w = _router(x2d, w_gate)
    slot1 = slots[:, 0].reshape(NW, TPW)
    slot2 = slots[:, 1].reshape(NW, TPW)
    te = te_w[:, 0]
    xg = _sc_dispatch()(xaug, slot1, slot2)
    y = _gmm(te, xg, w1, b1.reshape(N_EXP, 1, D_EXP), w2,
             b2.reshape(N_EXP, 1, D_MODEL))
    out = _sc_combine()(y, slot1, slot2)
    return out.reshape(x.shape)
